# Initial kernel scaffold; baseline (speedup 1.0000x reference)
import jax
import jax.numpy as jnp
from jax.experimental import pallas as pl


def kernel(edge_index, node_emb, W0, b0, g0, be0, rm0, rv0, W1, b1, g1, be1, rm1, rv1, We1, bE1, We2, bE2, We3, bE3, We4, bE4):
    raise NotImplementedError("write your pallas kernel here")


# speedup vs baseline: 1.2403x; 1.2403x over previous
import functools

import jax
import jax.numpy as jnp
from jax import lax
from jax.experimental import pallas as pl
from jax.experimental.pallas import tpu as pltpu
from jax.experimental.pallas import tpu_sc as plsc

_RB = 512
_EB = 1024


def _prep_body(x_ref, h_ref, w_ref, u_ref, dinv_ref):
    deg = h_ref[0][:, 0:1] + h_ref[1][:, 0:1] + 1.0
    dinv = lax.rsqrt(deg)
    y = jnp.dot(x_ref[...], w_ref[...], preferred_element_type=jnp.float32)
    u = y * dinv
    u_ref[0] = u[:, :32]
    u_ref[1] = u[:, 32:]
    dinv_ref[...] = dinv


def _prep_call(x, hist, w, np_):
    grid = (np_ // _RB,)
    return pl.pallas_call(
        _prep_body,
        grid=grid,
        in_specs=[
            pl.BlockSpec((_RB, 64), lambda i: (i, 0)),
            pl.BlockSpec((2, _RB, 16), lambda i: (0, i, 0)),
            pl.BlockSpec((64, 64), lambda i: (0, 0)),
        ],
        out_specs=[
            pl.BlockSpec((2, _RB, 32), lambda i: (0, i, 0)),
            pl.BlockSpec((_RB, 1), lambda i: (i, 0)),
        ],
        out_shape=[
            jax.ShapeDtypeStruct((2, np_, 32), jnp.float32),
            jax.ShapeDtypeStruct((np_, 1), jnp.float32),
        ],
    )(x, hist, w)


def _mid_body(acc_ref, u_ref, dinv_ref, s_ref, t_ref, w_ref, uo_ref):
    acc = jnp.concatenate([acc_ref[0], acc_ref[1]], axis=1)
    u = jnp.concatenate([u_ref[0], u_ref[1]], axis=1)
    x1 = jnp.maximum((acc + u) * dinv_ref[...] * s_ref[...] + t_ref[...], 0.0)
    y = jnp.dot(x1, w_ref[...], preferred_element_type=jnp.float32)
    u1 = y * dinv_ref[...]
    uo_ref[0] = u1[:, :32]
    uo_ref[1] = u1[:, 32:]


def _mid_call(acc, u, dinv, s, t, w, np_):
    grid = (np_ // _RB,)
    return pl.pallas_call(
        _mid_body,
        grid=grid,
        in_specs=[
            pl.BlockSpec((2, _RB, 32), lambda i: (0, i, 0)),
            pl.BlockSpec((2, _RB, 32), lambda i: (0, i, 0)),
            pl.BlockSpec((_RB, 1), lambda i: (i, 0)),
            pl.BlockSpec((1, 64), lambda i: (0, 0)),
            pl.BlockSpec((1, 64), lambda i: (0, 0)),
            pl.BlockSpec((64, 64), lambda i: (0, 0)),
        ],
        out_specs=pl.BlockSpec((2, _RB, 32), lambda i: (0, i, 0)),
        out_shape=jax.ShapeDtypeStruct((2, np_, 32), jnp.float32),
    )(acc, u, dinv, s, t, w)


def _fin_body(acc_ref, u_ref, dinv_ref, s_ref, t_ref, xo_ref):
    acc = jnp.concatenate([acc_ref[0], acc_ref[1]], axis=1)
    u = jnp.concatenate([u_ref[0], u_ref[1]], axis=1)
    xo_ref[...] = jnp.maximum(
        (acc + u) * dinv_ref[...] * s_ref[...] + t_ref[...], 0.0)


def _fin_call(acc, u, dinv, s, t, np_):
    grid = (np_ // _RB,)
    return pl.pallas_call(
        _fin_body,
        grid=grid,
        in_specs=[
            pl.BlockSpec((2, _RB, 32), lambda i: (0, i, 0)),
            pl.BlockSpec((2, _RB, 32), lambda i: (0, i, 0)),
            pl.BlockSpec((_RB, 1), lambda i: (i, 0)),
            pl.BlockSpec((1, 64), lambda i: (0, 0)),
            pl.BlockSpec((1, 64), lambda i: (0, 0)),
        ],
        out_specs=pl.BlockSpec((_RB, 64), lambda i: (i, 0)),
        out_shape=jax.ShapeDtypeStruct((np_, 64), jnp.float32),
    )(acc, u, dinv, s, t)


def _mlp_body(e_ref, w1_ref, b1_ref, w2_ref, b2_ref, w3_ref, b3_ref,
              w4_ref, b4_ref, o_ref):
    e = jnp.concatenate([e_ref[0], e_ref[1]], axis=1)
    h = jnp.maximum(
        jnp.dot(e, w1_ref[...], preferred_element_type=jnp.float32)
        + b1_ref[...], 0.0)
    h = jnp.maximum(
        jnp.dot(h, w2_ref[...], preferred_element_type=jnp.float32)
        + b2_ref[...], 0.0)
    h = jnp.maximum(
        jnp.dot(h, w3_ref[...], preferred_element_type=jnp.float32)
        + b3_ref[...], 0.0)
    o_ref[...] = jnp.sum(h * w4_ref[...], axis=1, keepdims=True) + b4_ref[...]


def _mlp_call(ed, w1, b1, w2, b2, w3, b3, w4, b4, ep):
    grid = (ep // _EB,)
    full = lambda shape: pl.BlockSpec(shape, lambda i: tuple(0 for _ in shape))
    return pl.pallas_call(
        _mlp_body,
        grid=grid,
        in_specs=[
            pl.BlockSpec((2, _EB, 64), lambda i: (0, i, 0)),
            full((128, 128)), full((1, 128)),
            full((128, 64)), full((1, 64)),
            full((64, 32)), full((1, 32)),
            full((1, 32)), full((1, 1)),
        ],
        out_specs=pl.BlockSpec((_EB, 1), lambda i: (i, 0)),
        out_shape=jax.ShapeDtypeStruct((ep, 1), jnp.float32),
    )(ed, w1, b1, w2, b2, w3, b3, w4, b4)


def _hist_stage(colp, np_):
    cnt = jnp.zeros((np_,), jnp.float32).at[colp].add(1.0)
    hist = jnp.zeros((2, np_, 16), jnp.float32).at[0, :, 0].set(cnt)
    return hist


def _msgpass_stage(u, rowp, colp, np_):
    acc0 = jnp.zeros((np_, 32), jnp.float32).at[colp].add(u[0][rowp])
    acc1 = jnp.zeros((np_, 32), jnp.float32).at[colp].add(u[1][rowp])
    return jnp.stack([acc0, acc1])


def _gather_stage(x2, rowp, colp):
    return jnp.stack([x2[rowp], x2[colp]])


def kernel(edge_index, node_emb, W0, b0, g0, be0, rm0, rv0,
           W1, b1, g1, be1, rm1, rv1,
           We1, bE1, We2, bE2, We3, bE3, We4, bE4):
    n, h = node_emb.shape
    e = edge_index.shape[1]
    np_ = ((n + 1 + _RB - 1) // _RB) * _RB
    ep = ((e + 2047) // 2048) * 2048

    rowp = jnp.concatenate(
        [edge_index[0], jnp.full((ep - e,), n, jnp.int32)])
    colp = jnp.concatenate(
        [edge_index[1], jnp.full((ep - e,), n, jnp.int32)])
    xpad = jnp.concatenate(
        [node_emb, jnp.zeros((np_ - n, h), jnp.float32)])

    eps = 1e-5
    s0 = (g0 / jnp.sqrt(rv0 + eps)).reshape(1, h)
    t0 = ((b0 - rm0) * s0[0] + be0).reshape(1, h)
    s1 = (g1 / jnp.sqrt(rv1 + eps)).reshape(1, h)
    t1 = ((b1 - rm1) * s1[0] + be1).reshape(1, h)

    hist = _hist_stage(colp, np_)
    u0, dinv = _prep_call(xpad, hist, W0, np_)
    acc0 = _msgpass_stage(u0, rowp, colp, np_)
    u1 = _mid_call(acc0, u0, dinv, s0, t0, W1, np_)
    acc1 = _msgpass_stage(u1, rowp, colp, np_)
    x2 = _fin_call(acc1, u1, dinv, s1, t1, np_)
    ed = _gather_stage(x2, rowp, colp)
    out = _mlp_call(ed, We1, bE1.reshape(1, 2 * h),
                    We2, bE2.reshape(1, h),
                    We3, bE3.reshape(1, h // 2),
                    We4.reshape(1, h // 2), bE4.reshape(1, 1), ep)
    return out[:e]


# speedup vs baseline: 6.2327x; 5.0250x over previous
import functools

import jax
import jax.numpy as jnp
from jax import lax
from jax.experimental import pallas as pl
from jax.experimental.pallas import tpu as pltpu
from jax.experimental.pallas import tpu_sc as plsc

_RB = 512
_EB = 1024


def _prep_body(x_ref, h_ref, w_ref, u_ref, dinv_ref):
    deg = h_ref[0][:, 0:1] + h_ref[1][:, 0:1] + 1.0
    dinv = lax.rsqrt(deg)
    y = jnp.dot(x_ref[...], w_ref[...], preferred_element_type=jnp.float32)
    u = y * dinv
    u_ref[0] = u[:, :32]
    u_ref[1] = u[:, 32:]
    dinv_ref[...] = dinv


def _prep_call(x, hist, w, np_):
    grid = (np_ // _RB,)
    return pl.pallas_call(
        _prep_body,
        grid=grid,
        in_specs=[
            pl.BlockSpec((_RB, 64), lambda i: (i, 0)),
            pl.BlockSpec((2, _RB, 16), lambda i: (0, i, 0)),
            pl.BlockSpec((64, 64), lambda i: (0, 0)),
        ],
        out_specs=[
            pl.BlockSpec((2, _RB, 32), lambda i: (0, i, 0)),
            pl.BlockSpec((_RB, 1), lambda i: (i, 0)),
        ],
        out_shape=[
            jax.ShapeDtypeStruct((2, np_, 32), jnp.float32),
            jax.ShapeDtypeStruct((np_, 1), jnp.float32),
        ],
    )(x, hist, w)


def _mid_body(acc_ref, u_ref, dinv_ref, s_ref, t_ref, w_ref, uo_ref):
    acc = jnp.concatenate([acc_ref[0], acc_ref[1]], axis=1)
    u = jnp.concatenate([u_ref[0], u_ref[1]], axis=1)
    x1 = jnp.maximum((acc + u) * dinv_ref[...] * s_ref[...] + t_ref[...], 0.0)
    y = jnp.dot(x1, w_ref[...], preferred_element_type=jnp.float32)
    u1 = y * dinv_ref[...]
    uo_ref[0] = u1[:, :32]
    uo_ref[1] = u1[:, 32:]


def _mid_call(acc, u, dinv, s, t, w, np_):
    grid = (np_ // _RB,)
    return pl.pallas_call(
        _mid_body,
        grid=grid,
        in_specs=[
            pl.BlockSpec((2, _RB, 32), lambda i: (0, i, 0)),
            pl.BlockSpec((2, _RB, 32), lambda i: (0, i, 0)),
            pl.BlockSpec((_RB, 1), lambda i: (i, 0)),
            pl.BlockSpec((1, 64), lambda i: (0, 0)),
            pl.BlockSpec((1, 64), lambda i: (0, 0)),
            pl.BlockSpec((64, 64), lambda i: (0, 0)),
        ],
        out_specs=pl.BlockSpec((2, _RB, 32), lambda i: (0, i, 0)),
        out_shape=jax.ShapeDtypeStruct((2, np_, 32), jnp.float32),
    )(acc, u, dinv, s, t, w)


def _fin_body(acc_ref, u_ref, dinv_ref, s_ref, t_ref, xo_ref):
    acc = jnp.concatenate([acc_ref[0], acc_ref[1]], axis=1)
    u = jnp.concatenate([u_ref[0], u_ref[1]], axis=1)
    xo_ref[...] = jnp.maximum(
        (acc + u) * dinv_ref[...] * s_ref[...] + t_ref[...], 0.0)


def _fin_call(acc, u, dinv, s, t, np_):
    grid = (np_ // _RB,)
    return pl.pallas_call(
        _fin_body,
        grid=grid,
        in_specs=[
            pl.BlockSpec((2, _RB, 32), lambda i: (0, i, 0)),
            pl.BlockSpec((2, _RB, 32), lambda i: (0, i, 0)),
            pl.BlockSpec((_RB, 1), lambda i: (i, 0)),
            pl.BlockSpec((1, 64), lambda i: (0, 0)),
            pl.BlockSpec((1, 64), lambda i: (0, 0)),
        ],
        out_specs=pl.BlockSpec((_RB, 64), lambda i: (i, 0)),
        out_shape=jax.ShapeDtypeStruct((np_, 64), jnp.float32),
    )(acc, u, dinv, s, t)


def _mlp_body(e_ref, w1_ref, b1_ref, w2_ref, b2_ref, w3_ref, b3_ref,
              w4_ref, b4_ref, o_ref):
    e = jnp.concatenate([e_ref[0], e_ref[1]], axis=1)
    h = jnp.maximum(
        jnp.dot(e, w1_ref[...], preferred_element_type=jnp.float32)
        + b1_ref[...], 0.0)
    h = jnp.maximum(
        jnp.dot(h, w2_ref[...], preferred_element_type=jnp.float32)
        + b2_ref[...], 0.0)
    h = jnp.maximum(
        jnp.dot(h, w3_ref[...], preferred_element_type=jnp.float32)
        + b3_ref[...], 0.0)
    o_ref[...] = jnp.sum(h * w4_ref[...], axis=1, keepdims=True) + b4_ref[...]


def _mlp_call(ed, w1, b1, w2, b2, w3, b3, w4, b4, ep):
    grid = (ep // _EB,)
    full = lambda shape: pl.BlockSpec(shape, lambda i: tuple(0 for _ in shape))
    return pl.pallas_call(
        _mlp_body,
        grid=grid,
        in_specs=[
            pl.BlockSpec((2, _EB, 64), lambda i: (0, i, 0)),
            full((128, 128)), full((1, 128)),
            full((128, 64)), full((1, 64)),
            full((64, 32)), full((1, 32)),
            full((1, 32)), full((1, 1)),
        ],
        out_specs=pl.BlockSpec((_EB, 1), lambda i: (i, 0)),
        out_shape=jax.ShapeDtypeStruct((ep, 1), jnp.float32),
    )(ed, w1, b1, w2, b2, w3, b3, w4, b4)


_MESH = plsc.VectorSubcoreMesh(core_axis_name="c", subcore_axis_name="s")
_SC_PARAMS = pltpu.CompilerParams(use_tc_tiling_on_sc=False)
_NS = 16
_CH = 20
_K = 4


def _hist_call(col2, np_):
    nb2 = col2.shape[1]
    bpt = nb2 // _NS
    rpt = np_ // _NS

    @functools.partial(
        pl.kernel,
        out_type=jax.ShapeDtypeStruct((2 * np_, 16), jnp.float32),
        mesh=_MESH,
        compiler_params=_SC_PARAMS,
        scratch_types=[
            pltpu.VMEM((392, 16), jnp.float32),
            pltpu.VMEM((bpt, 128), jnp.int32),
            pltpu.VMEM((128, 16), jnp.float32),
            pltpu.VMEM_SHARED((np_, 16), jnp.float32),
            pltpu.SemaphoreType.DMA((_K,)),
        ],
    )
    def k(col_hbm, out_hbm, zbuf, idx_v, ones_v, acc, sems):
        c = lax.axis_index("c")
        s = lax.axis_index("s")

        @pl.loop(0, 392)
        def _(i):
            zbuf[i, :] = jnp.zeros((16,), jnp.float32)

        @pl.loop(0, 128)
        def _(i):
            ones_v[i, :] = jnp.ones((16,), jnp.float32)

        @pl.loop(0, rpt, step=392)
        def _(r):
            pltpu.sync_copy(zbuf, acc.at[pl.ds(s * rpt + r, 392)])

        plsc.subcore_barrier()
        pltpu.sync_copy(col_hbm.at[c, pl.ds(s * bpt, bpt)], idx_v)

        @pl.loop(0, bpt, step=_K)
        def _(j):
            ds_ = [pltpu.async_copy(ones_v, acc.at[idx_v.at[j + kk]],
                                    sems.at[kk], add=True)
                   for kk in range(_K)]
            for d in ds_:
                d.wait()

        plsc.subcore_barrier()
        pltpu.sync_copy(acc.at[pl.ds(s * rpt, rpt)],
                        out_hbm.at[pl.ds(c * np_ + s * rpt, rpt)])

    return k(col2)


def _msgpass_call(uflat, rowslab, col2d, np_):
    nb = col2d.shape[0]
    bpt = nb // _NS
    rpt = np_ // _NS

    @functools.partial(
        pl.kernel,
        out_type=jax.ShapeDtypeStruct((2 * np_, 32), jnp.float32),
        mesh=_MESH,
        compiler_params=_SC_PARAMS,
        scratch_types=[
            pltpu.VMEM((98, 32), jnp.float32),
            pltpu.VMEM((_CH, 128), jnp.int32),
            pltpu.VMEM((_CH, 128), jnp.int32),
            pltpu.VMEM((_K, 128, 32), jnp.float32),
            pltpu.VMEM_SHARED((np_, 32), jnp.float32),
            pltpu.SemaphoreType.DMA((_K,)),
            pltpu.SemaphoreType.DMA((_K,)),
        ],
    )
    def k(u_hbm, row_hbm, colk_hbm, out_hbm,
          zbuf, idxr, idxc, bufs, acc, gsems, wsems):
        c = lax.axis_index("c")
        s = lax.axis_index("s")

        @pl.loop(0, 98)
        def _(i):
            zbuf[i, pl.ds(0, 16)] = jnp.zeros((16,), jnp.float32)
            zbuf[i, pl.ds(16, 16)] = jnp.zeros((16,), jnp.float32)

        @pl.loop(0, rpt, step=98)
        def _(r):
            pltpu.sync_copy(zbuf, acc.at[pl.ds(s * rpt + r, 98)])

        plsc.subcore_barrier()

        @pl.loop(0, bpt, step=_CH)
        def _(ii):
            pltpu.sync_copy(row_hbm.at[c, pl.ds(s * bpt + ii, _CH)], idxr)
            pltpu.sync_copy(colk_hbm.at[pl.ds(s * bpt + ii, _CH)], idxc)

            @pl.loop(0, _CH, step=_K)
            def _(j):
                gs = [pltpu.async_copy(u_hbm.at[idxr.at[j + kk]],
                                       bufs.at[kk], gsems.at[kk])
                      for kk in range(_K)]
                ws = []
                for kk in range(_K):
                    gs[kk].wait()
                    ws.append(pltpu.async_copy(
                        bufs.at[kk], acc.at[idxc.at[j + kk]],
                        wsems.at[kk], add=True))
                for w in ws:
                    w.wait()

        plsc.subcore_barrier()
        pltpu.sync_copy(acc.at[pl.ds(s * rpt, rpt)],
                        out_hbm.at[pl.ds(c * np_ + s * rpt, rpt)])

    return k(uflat, rowslab, col2d)


def _gather_call(x2, idx2, ep):
    nb = idx2.shape[1]
    bpt = nb // _NS

    @functools.partial(
        pl.kernel,
        out_type=jax.ShapeDtypeStruct((2 * ep, 64), jnp.float32),
        mesh=_MESH,
        compiler_params=_SC_PARAMS,
        scratch_types=[
            pltpu.VMEM((_CH, 128), jnp.int32),
            pltpu.VMEM((_K, 128, 64), jnp.float32),
            pltpu.SemaphoreType.DMA((_K,)),
            pltpu.SemaphoreType.DMA((_K,)),
        ],
    )
    def k(x_hbm, idx_hbm, out_hbm, idx_v, bufs, gsems, wsems):
        c = lax.axis_index("c")
        s = lax.axis_index("s")

        @pl.loop(0, bpt, step=_CH)
        def _(ii):
            pltpu.sync_copy(idx_hbm.at[c, pl.ds(s * bpt + ii, _CH)], idx_v)

            @pl.loop(0, _CH, step=_K)
            def _(j):
                gs = [pltpu.async_copy(x_hbm.at[idx_v.at[j + kk]],
                                       bufs.at[kk], gsems.at[kk])
                      for kk in range(_K)]
                ws = []
                for kk in range(_K):
                    gs[kk].wait()
                    base = (c * nb + s * bpt + ii + j + kk) * 128
                    ws.append(pltpu.async_copy(
                        bufs.at[kk], out_hbm.at[pl.ds(base, 128)],
                        wsems.at[kk]))
                for w in ws:
                    w.wait()

    return k(x2, idx2)


def kernel(edge_index, node_emb, W0, b0, g0, be0, rm0, rv0,
           W1, b1, g1, be1, rm1, rv1,
           We1, bE1, We2, bE2, We3, bE3, We4, bE4):
    n, h = node_emb.shape
    e = edge_index.shape[1]
    np_ = ((n + 1 + _RB - 1) // _RB) * _RB
    ep = ((e + 32767) // 32768) * 32768

    rowp = jnp.concatenate(
        [edge_index[0], jnp.full((ep - e,), n, jnp.int32)])
    colp = jnp.concatenate(
        [edge_index[1], jnp.full((ep - e,), n, jnp.int32)])
    xpad = jnp.concatenate(
        [node_emb, jnp.zeros((np_ - n, h), jnp.float32)])

    eps = 1e-5
    s0 = (g0 / jnp.sqrt(rv0 + eps)).reshape(1, h)
    t0 = ((b0 - rm0) * s0[0] + be0).reshape(1, h)
    s1 = (g1 / jnp.sqrt(rv1 + eps)).reshape(1, h)
    t1 = ((b1 - rm1) * s1[0] + be1).reshape(1, h)

    nb = ep // 128
    row2d = rowp.reshape(nb, 128)
    col2d = colp.reshape(nb, 128)
    col2 = colp.reshape(2, nb // 2, 128)
    rowslab = jnp.stack([row2d, row2d + np_])
    idx2 = jnp.stack([row2d, col2d])

    hist = _hist_call(col2, np_).reshape(2, np_, 16)
    u0, dinv = _prep_call(xpad, hist, W0, np_)
    acc0 = _msgpass_call(u0.reshape(2 * np_, 32), rowslab,
                         col2d, np_).reshape(2, np_, 32)
    u1 = _mid_call(acc0, u0, dinv, s0, t0, W1, np_)
    acc1 = _msgpass_call(u1.reshape(2 * np_, 32), rowslab,
                         col2d, np_).reshape(2, np_, 32)
    x2 = _fin_call(acc1, u1, dinv, s1, t1, np_)
    ed = _gather_call(x2, idx2, ep).reshape(2, ep, 64)
    out = _mlp_call(ed, We1, bE1.reshape(1, 2 * h),
                    We2, bE2.reshape(1, h),
                    We3, bE3.reshape(1, h // 2),
                    We4.reshape(1, h // 2), bE4.reshape(1, 1), ep)
    return out[:e]


# speedup vs baseline: 6.4896x; 1.0412x over previous
import functools

import jax
import jax.numpy as jnp
from jax import lax
from jax.experimental import pallas as pl
from jax.experimental.pallas import tpu as pltpu
from jax.experimental.pallas import tpu_sc as plsc

_RB = 512
_EB = 1024


def _prep_body(x_ref, h_ref, w_ref, u_ref, dinv_ref):
    deg = h_ref[0][:, 0:1] + h_ref[1][:, 0:1] + 1.0
    dinv = lax.rsqrt(deg)
    y = jnp.dot(x_ref[...], w_ref[...], preferred_element_type=jnp.float32)
    u = y * dinv
    u_ref[0] = u[:, :32]
    u_ref[1] = u[:, 32:]
    dinv_ref[...] = dinv


def _prep_call(x, hist, w, np_):
    grid = (np_ // _RB,)
    return pl.pallas_call(
        _prep_body,
        grid=grid,
        in_specs=[
            pl.BlockSpec((_RB, 64), lambda i: (i, 0)),
            pl.BlockSpec((2, _RB, 16), lambda i: (0, i, 0)),
            pl.BlockSpec((64, 64), lambda i: (0, 0)),
        ],
        out_specs=[
            pl.BlockSpec((2, _RB, 32), lambda i: (0, i, 0)),
            pl.BlockSpec((_RB, 1), lambda i: (i, 0)),
        ],
        out_shape=[
            jax.ShapeDtypeStruct((2, np_, 32), jnp.float32),
            jax.ShapeDtypeStruct((np_, 1), jnp.float32),
        ],
    )(x, hist, w)


def _mid_body(acc_ref, u_ref, dinv_ref, s_ref, t_ref, w_ref, uo_ref):
    acc = jnp.concatenate([acc_ref[0], acc_ref[1]], axis=1)
    u = jnp.concatenate([u_ref[0], u_ref[1]], axis=1)
    x1 = jnp.maximum((acc + u) * dinv_ref[...] * s_ref[...] + t_ref[...], 0.0)
    y = jnp.dot(x1, w_ref[...], preferred_element_type=jnp.float32)
    u1 = y * dinv_ref[...]
    uo_ref[0] = u1[:, :32]
    uo_ref[1] = u1[:, 32:]


def _mid_call(acc, u, dinv, s, t, w, np_):
    grid = (np_ // _RB,)
    return pl.pallas_call(
        _mid_body,
        grid=grid,
        in_specs=[
            pl.BlockSpec((2, _RB, 32), lambda i: (0, i, 0)),
            pl.BlockSpec((2, _RB, 32), lambda i: (0, i, 0)),
            pl.BlockSpec((_RB, 1), lambda i: (i, 0)),
            pl.BlockSpec((1, 64), lambda i: (0, 0)),
            pl.BlockSpec((1, 64), lambda i: (0, 0)),
            pl.BlockSpec((64, 64), lambda i: (0, 0)),
        ],
        out_specs=pl.BlockSpec((2, _RB, 32), lambda i: (0, i, 0)),
        out_shape=jax.ShapeDtypeStruct((2, np_, 32), jnp.float32),
    )(acc, u, dinv, s, t, w)


def _fin_body(acc_ref, u_ref, dinv_ref, s_ref, t_ref, xo_ref):
    acc = jnp.concatenate([acc_ref[0], acc_ref[1]], axis=1)
    u = jnp.concatenate([u_ref[0], u_ref[1]], axis=1)
    xo_ref[...] = jnp.maximum(
        (acc + u) * dinv_ref[...] * s_ref[...] + t_ref[...],
        0.0).astype(jnp.bfloat16)


def _fin_call(acc, u, dinv, s, t, np_):
    grid = (np_ // _RB,)
    return pl.pallas_call(
        _fin_body,
        grid=grid,
        in_specs=[
            pl.BlockSpec((2, _RB, 32), lambda i: (0, i, 0)),
            pl.BlockSpec((2, _RB, 32), lambda i: (0, i, 0)),
            pl.BlockSpec((_RB, 1), lambda i: (i, 0)),
            pl.BlockSpec((1, 64), lambda i: (0, 0)),
            pl.BlockSpec((1, 64), lambda i: (0, 0)),
        ],
        out_specs=pl.BlockSpec((_RB, 64), lambda i: (i, 0)),
        out_shape=jax.ShapeDtypeStruct((np_, 64), jnp.bfloat16),
    )(acc, u, dinv, s, t)


def _mlp_body(e_ref, w1_ref, b1_ref, w2_ref, b2_ref, w3_ref, b3_ref,
              w4_ref, b4_ref, o_ref):
    e = jnp.concatenate([e_ref[0], e_ref[1]], axis=1)
    h = jnp.maximum(
        jnp.dot(e, w1_ref[...], preferred_element_type=jnp.float32)
        + b1_ref[...], 0.0).astype(jnp.bfloat16)
    h = jnp.maximum(
        jnp.dot(h, w2_ref[...], preferred_element_type=jnp.float32)
        + b2_ref[...], 0.0).astype(jnp.bfloat16)
    h = jnp.maximum(
        jnp.dot(h, w3_ref[...], preferred_element_type=jnp.float32)
        + b3_ref[...], 0.0)
    o_ref[...] = jnp.sum(h * w4_ref[...], axis=1, keepdims=True) + b4_ref[...]


def _mlp_call(ed, w1, b1, w2, b2, w3, b3, w4, b4, ep):
    grid = (ep // _EB,)
    full = lambda shape: pl.BlockSpec(shape, lambda i: tuple(0 for _ in shape))
    return pl.pallas_call(
        _mlp_body,
        grid=grid,
        in_specs=[
            pl.BlockSpec((2, _EB, 64), lambda i: (0, i, 0)),
            full((128, 128)), full((1, 128)),
            full((128, 64)), full((1, 64)),
            full((64, 32)), full((1, 32)),
            full((1, 32)), full((1, 1)),
        ],
        out_specs=pl.BlockSpec((_EB, 1), lambda i: (i, 0)),
        out_shape=jax.ShapeDtypeStruct((ep, 1), jnp.float32),
    )(ed, w1, b1, w2, b2, w3, b3, w4, b4)


_MESH = plsc.VectorSubcoreMesh(core_axis_name="c", subcore_axis_name="s")
_SC_PARAMS = pltpu.CompilerParams(use_tc_tiling_on_sc=False)
_NS = 16
_CH = 20
_K = 4


def _hist_call(col2, np_):
    nb2 = col2.shape[1]
    bpt = nb2 // _NS
    rpt = np_ // _NS

    @functools.partial(
        pl.kernel,
        out_type=jax.ShapeDtypeStruct((2 * np_, 16), jnp.float32),
        mesh=_MESH,
        compiler_params=_SC_PARAMS,
        scratch_types=[
            pltpu.VMEM((392, 16), jnp.float32),
            pltpu.VMEM((bpt, 128), jnp.int32),
            pltpu.VMEM((128, 16), jnp.float32),
            pltpu.VMEM_SHARED((np_, 16), jnp.float32),
            pltpu.SemaphoreType.DMA((_K,)),
        ],
    )
    def k(col_hbm, out_hbm, zbuf, idx_v, ones_v, acc, sems):
        c = lax.axis_index("c")
        s = lax.axis_index("s")

        @pl.loop(0, 392)
        def _(i):
            zbuf[i, :] = jnp.zeros((16,), jnp.float32)

        @pl.loop(0, 128)
        def _(i):
            ones_v[i, :] = jnp.ones((16,), jnp.float32)

        @pl.loop(0, rpt, step=392)
        def _(r):
            pltpu.sync_copy(zbuf, acc.at[pl.ds(s * rpt + r, 392)])

        plsc.subcore_barrier()
        pltpu.sync_copy(col_hbm.at[c, pl.ds(s * bpt, bpt)], idx_v)

        @pl.loop(0, bpt, step=_K)
        def _(j):
            ds_ = [pltpu.async_copy(ones_v, acc.at[idx_v.at[j + kk]],
                                    sems.at[kk], add=True)
                   for kk in range(_K)]
            for d in ds_:
                d.wait()

        plsc.subcore_barrier()
        pltpu.sync_copy(acc.at[pl.ds(s * rpt, rpt)],
                        out_hbm.at[pl.ds(c * np_ + s * rpt, rpt)])

    return k(col2)


def _msgpass_call(uflat, rowslab, col2d, np_):
    nb = col2d.shape[0]
    bpt = nb // _NS
    rpt = np_ // _NS

    @functools.partial(
        pl.kernel,
        out_type=jax.ShapeDtypeStruct((2 * np_, 32), jnp.float32),
        mesh=_MESH,
        compiler_params=_SC_PARAMS,
        scratch_types=[
            pltpu.VMEM((_CH, 128), jnp.int32),
            pltpu.VMEM((_CH, 128), jnp.int32),
            pltpu.VMEM((_K * 128, 32), jnp.float32),
            pltpu.VMEM_SHARED((np_, 32), jnp.float32),
            pltpu.SemaphoreType.DMA((_K,)),
            pltpu.SemaphoreType.DMA((_K,)),
        ],
    )
    def k(u_hbm, row_hbm, colk_hbm, out_hbm,
          idxr, idxc, buf, acc, gsems, wsems):
        c = lax.axis_index("c")
        s = lax.axis_index("s")

        @pl.loop(0, _K * 128)
        def _(i):
            buf[i, pl.ds(0, 16)] = jnp.zeros((16,), jnp.float32)
            buf[i, pl.ds(16, 16)] = jnp.zeros((16,), jnp.float32)

        nz = _K * 128
        @pl.loop(0, rpt - (rpt % nz), step=nz)
        def _(r):
            pltpu.sync_copy(buf, acc.at[pl.ds(s * rpt + r, nz)])

        if rpt % nz:
            pltpu.sync_copy(buf.at[pl.ds(0, rpt % nz)],
                            acc.at[pl.ds(s * rpt + rpt - (rpt % nz),
                                         rpt % nz)])

        plsc.subcore_barrier()

        def slot(kk):
            return buf.at[pl.ds(kk * 128, 128)]

        @pl.loop(0, bpt, step=_CH)
        def _(ii):
            pltpu.sync_copy(row_hbm.at[c, pl.ds(s * bpt + ii, _CH)], idxr)
            pltpu.sync_copy(colk_hbm.at[pl.ds(s * bpt + ii, _CH)], idxc)
            for kk in range(_K):
                pltpu.async_copy(u_hbm.at[idxr.at[kk]], slot(kk),
                                 gsems.at[kk])

            @pl.loop(0, _CH - _K, step=_K)
            def _(j):
                for kk in range(_K):
                    pltpu.make_async_copy(u_hbm.at[idxr.at[0]], slot(kk),
                                          gsems.at[kk]).wait()
                    pltpu.async_copy(slot(kk), acc.at[idxc.at[j + kk]],
                                     wsems.at[kk], add=True)
                for kk in range(_K):
                    pltpu.make_async_copy(slot(kk), acc.at[idxc.at[0]],
                                          wsems.at[kk]).wait()
                    pltpu.async_copy(u_hbm.at[idxr.at[j + _K + kk]],
                                     slot(kk), gsems.at[kk])

            for kk in range(_K):
                pltpu.make_async_copy(u_hbm.at[idxr.at[0]], slot(kk),
                                      gsems.at[kk]).wait()
                pltpu.async_copy(slot(kk), acc.at[idxc.at[_CH - _K + kk]],
                                 wsems.at[kk], add=True)
            for kk in range(_K):
                pltpu.make_async_copy(slot(kk), acc.at[idxc.at[0]],
                                      wsems.at[kk]).wait()

        plsc.subcore_barrier()
        pltpu.sync_copy(acc.at[pl.ds(s * rpt, rpt)],
                        out_hbm.at[pl.ds(c * np_ + s * rpt, rpt)])

    return k(uflat, rowslab, col2d)


def _gather_call(x2, idx2, ep):
    nb = idx2.shape[1]
    bpt = nb // _NS
    gpb = 8
    ngr = bpt // gpb

    @functools.partial(
        pl.kernel,
        out_type=jax.ShapeDtypeStruct((2 * ep, 64), jnp.bfloat16),
        mesh=_MESH,
        compiler_params=_SC_PARAMS,
        scratch_types=[
            pltpu.VMEM((bpt, 128), jnp.int32),
            pltpu.VMEM((2, gpb * 128, 64), jnp.bfloat16),
            pltpu.SemaphoreType.DMA((2,)),
            pltpu.SemaphoreType.DMA((2,)),
        ],
    )
    def k(x_hbm, idx_hbm, out_hbm, idx_v, bufs, gsems, wsems):
        c = lax.axis_index("c")
        s = lax.axis_index("s")
        pltpu.sync_copy(idx_hbm.at[c, pl.ds(s * bpt, bpt)], idx_v)
        base0 = (c * nb + s * bpt) * 128

        def gfire(g, p):
            for kk in range(gpb):
                pltpu.async_copy(x_hbm.at[idx_v.at[g * gpb + kk]],
                                 bufs.at[p, pl.ds(kk * 128, 128)],
                                 gsems.at[p])

        def gdrain(p):
            pltpu.make_async_copy(x_hbm.at[pl.ds(0, gpb * 128)],
                                  bufs.at[p], gsems.at[p]).wait()

        def wfire(g, p):
            pltpu.async_copy(bufs.at[p],
                             out_hbm.at[pl.ds(base0 + g * gpb * 128,
                                              gpb * 128)],
                             wsems.at[p])

        def wdrain(p):
            pltpu.make_async_copy(bufs.at[p],
                                  out_hbm.at[pl.ds(base0, gpb * 128)],
                                  wsems.at[p]).wait()

        gfire(0, 0)
        gfire(1, 1)

        @pl.loop(0, ngr - 2, step=2)
        def _(g):
            for p in range(2):
                gdrain(p)
                wfire(g + p, p)
                wdrain(p)
                gfire(g + p + 2, p)

        for p in range(2):
            gdrain(p)
            wfire(ngr - 2 + p, p)
        for p in range(2):
            wdrain(p)

    return k(x2, idx2)


def kernel(edge_index, node_emb, W0, b0, g0, be0, rm0, rv0,
           W1, b1, g1, be1, rm1, rv1,
           We1, bE1, We2, bE2, We3, bE3, We4, bE4):
    n, h = node_emb.shape
    e = edge_index.shape[1]
    np_ = ((n + 1 + _RB - 1) // _RB) * _RB
    ep = ((e + 32767) // 32768) * 32768

    rowp = jnp.concatenate(
        [edge_index[0], jnp.full((ep - e,), n, jnp.int32)])
    colp = jnp.concatenate(
        [edge_index[1], jnp.full((ep - e,), n, jnp.int32)])
    xpad = jnp.concatenate(
        [node_emb, jnp.zeros((np_ - n, h), jnp.float32)])

    eps = 1e-5
    s0 = (g0 / jnp.sqrt(rv0 + eps)).reshape(1, h)
    t0 = ((b0 - rm0) * s0[0] + be0).reshape(1, h)
    s1 = (g1 / jnp.sqrt(rv1 + eps)).reshape(1, h)
    t1 = ((b1 - rm1) * s1[0] + be1).reshape(1, h)

    nb = ep // 128
    row2d = rowp.reshape(nb, 128)
    col2d = colp.reshape(nb, 128)
    col2 = colp.reshape(2, nb // 2, 128)
    rowslab = jnp.stack([row2d, row2d + np_])
    idx2 = jnp.stack([row2d, col2d])

    hist = _hist_call(col2, np_).reshape(2, np_, 16)
    u0, dinv = _prep_call(xpad, hist, W0, np_)
    acc0 = _msgpass_call(u0.reshape(2 * np_, 32), rowslab,
                         col2d, np_).reshape(2, np_, 32)
    u1 = _mid_call(acc0, u0, dinv, s0, t0, W1, np_)
    acc1 = _msgpass_call(u1.reshape(2 * np_, 32), rowslab,
                         col2d, np_).reshape(2, np_, 32)
    x2 = _fin_call(acc1, u1, dinv, s1, t1, np_)
    ed = _gather_call(x2, idx2, ep).reshape(2, ep, 64)
    out = _mlp_call(ed, We1.astype(jnp.bfloat16), bE1.reshape(1, 2 * h),
                    We2.astype(jnp.bfloat16), bE2.reshape(1, h),
                    We3.astype(jnp.bfloat16), bE3.reshape(1, h // 2),
                    We4.reshape(1, h // 2), bE4.reshape(1, 1), ep)
    return out[:e]
